# baseline (speedup 1.0000x reference)
import functools

import jax
import jax.numpy as jnp
from jax import lax
from jax.experimental import pallas as pl
from jax.experimental.pallas import tpu as pltpu
from jax.experimental.pallas import tpu_sc as plsc

N_NODES = 10000
D = 128
N_EDGES = 320000

_NC = 2
_NS = 16
_NW = _NC * _NS
_L = 16
_ROW_BLK = 2000

_CHUNK = 128
_BASE_CHUNKS = 78
_MAX_EDGES = (_BASE_CHUNKS + 1) * _CHUNK


_MV_BLK = 2048
_MV_N = 5


def _matvec_body(
    x_hbm, wrel_ref, wroot_ref, b_ref, srel_ref, base_ref, bufa, bufb, sema, semb
):
    dn = (((1,), (1,)), ((), ()))
    bufs = (bufa, bufb)
    sems = (sema, semb)

    def make_cp(i):
        n = min(_MV_BLK, N_NODES - i * _MV_BLK)
        return pltpu.make_async_copy(
            x_hbm.at[pl.ds(i * _MV_BLK, n)], bufs[i % 2].at[pl.ds(0, n)], sems[i % 2]
        )

    make_cp(0).start()
    make_cp(1).start()
    for i in range(_MV_N):
        n = min(_MV_BLK, N_NODES - i * _MV_BLK)
        make_cp(i).wait()
        xb = bufs[i % 2][...]
        srow = jax.lax.dot_general(
            wrel_ref[...], xb, dn, preferred_element_type=jnp.float32
        )
        brow = (
            jax.lax.dot_general(wroot_ref[...], xb, dn, preferred_element_type=jnp.float32)
            + b_ref[0, 0]
        )
        srel_ref[:, pl.ds(i * _MV_BLK, n)] = srow[:, :n]
        base_ref[:, pl.ds(i * _MV_BLK, n)] = brow[:, :n]
        if i + 2 < _MV_N:
            make_cp(i + 2).start()


def _edge_body(srel_hbm, edge_hbm, out_hbm, srel_sh, srel_v, edges_v, acc_v, sem):
    sid = lax.axis_index("s")
    wid = sid * _NC + lax.axis_index("c")
    extra = jnp.maximum(wid - 28, 0)
    c0 = pl.multiple_of((wid * _BASE_CHUNKS + extra) * _CHUNK, _CHUNK)
    nvec = (_BASE_CHUNKS * _CHUNK) // _L + jnp.where(wid >= 28, 8, 0)

    cp_e = pltpu.async_copy(edge_hbm.at[:, pl.ds(c0, _MAX_EDGES)], edges_v, sem)

    with jax.named_scope("ph_srel_bcast"):

        @pl.when(sid == 0)
        def _():
            pltpu.sync_copy(srel_hbm.at[0], srel_sh)

        plsc.subcore_barrier()
        pltpu.sync_copy(srel_sh, srel_v)

    zero16 = jnp.zeros((_L,), jnp.float32)

    with jax.named_scope("ph_zero"):

        @plsc.parallel_loop(0, N_NODES // _L, unroll=8)
        def zero_step(i):
            acc_v[pl.ds(i * _L, _L)] = zero16

    with jax.named_scope("ph_dma_wait"):
        cp_e.wait()

    with jax.named_scope("ph_edges"):

        @plsc.parallel_loop(0, nvec, unroll=8)
        def edge_step(k):
            sl = pl.ds(k * _L, _L)
            vals = plsc.load_gather(srel_v, [edges_v[0, sl]])
            plsc.addupdate_scatter(acc_v, [edges_v[1, sl]], vals)

    with jax.named_scope("ph_out"):
        pltpu.sync_copy(acc_v, out_hbm.at[wid])


_edge_kernel = functools.partial(
    pl.kernel,
    mesh=plsc.VectorSubcoreMesh(core_axis_name="c", subcore_axis_name="s"),
    compiler_params=pltpu.CompilerParams(needs_layout_passes=False),
    out_type=jax.ShapeDtypeStruct((_NW, N_NODES), jnp.float32),
    scratch_types=[
        pltpu.VMEM_SHARED((N_NODES,), jnp.float32),
        pltpu.VMEM((N_NODES,), jnp.float32),
        pltpu.VMEM((2, _MAX_EDGES), jnp.int32),
        pltpu.VMEM((N_NODES,), jnp.float32),
        pltpu.SemaphoreType.DMA,
    ],
)(_edge_body)


def _combine_body(p_ref, base_ref, out_ref):
    out_ref[...] = jnp.sum(p_ref[...], axis=0, keepdims=True) + base_ref[...]


def kernel(x, edge_index, W_rel, b_rel, W_root):
    edges = edge_index.astype(jnp.int32)
    srel, base = pl.pallas_call(
        _matvec_body,
        in_specs=[
            pl.BlockSpec(memory_space=pltpu.HBM),
            pl.BlockSpec((1, D), lambda: (0, 0)),
            pl.BlockSpec((1, D), lambda: (0, 0)),
            pl.BlockSpec((1, 1), lambda: (0, 0)),
        ],
        out_shape=[
            jax.ShapeDtypeStruct((1, N_NODES), jnp.float32),
            jax.ShapeDtypeStruct((1, N_NODES), jnp.float32),
        ],
        scratch_shapes=[
            pltpu.VMEM((_MV_BLK, D), jnp.float32),
            pltpu.VMEM((_MV_BLK, D), jnp.float32),
            pltpu.SemaphoreType.DMA,
            pltpu.SemaphoreType.DMA,
        ],
    )(x, W_rel.reshape(1, D), W_root.reshape(1, D), b_rel.reshape(1, 1))

    partials = _edge_kernel(srel, edges)

    out_row = pl.pallas_call(
        _combine_body,
        out_shape=jax.ShapeDtypeStruct((1, N_NODES), jnp.float32),
    )(partials, base)
    return out_row.reshape(N_NODES, 1)


# speedup vs baseline: 1.0435x; 1.0435x over previous
import functools

import jax
import jax.numpy as jnp
from jax import lax
from jax.experimental import pallas as pl
from jax.experimental.pallas import tpu as pltpu
from jax.experimental.pallas import tpu_sc as plsc

N_NODES = 10000
D = 128
N_EDGES = 320000

_NC = 2
_NS = 16
_NW = _NC * _NS
_L = 16
_ROW_BLK = 2000

_CHUNK = 128
_BASE_CHUNKS = 78
_MAX_EDGES = (_BASE_CHUNKS + 1) * _CHUNK


def _matvec_body(x_ref, wrel_ref, wroot_ref, b_ref, srel_ref, base_ref):
    xb = x_ref[...]
    dn = (((1,), (1,)), ((), ()))
    srel_ref[...] = jax.lax.dot_general(
        wrel_ref[...], xb, dn, preferred_element_type=jnp.float32
    )
    base_ref[...] = (
        jax.lax.dot_general(wroot_ref[...], xb, dn, preferred_element_type=jnp.float32)
        + b_ref[0, 0]
    )


def _edge_body(srel_hbm, edge_hbm, out_hbm, srel_sh, srel_v, edges_v, acc_v, sem):
    sid = lax.axis_index("s")
    wid = sid * _NC + lax.axis_index("c")
    extra = jnp.maximum(wid - 28, 0)
    c0 = pl.multiple_of((wid * _BASE_CHUNKS + extra) * _CHUNK, _CHUNK)
    nvec = (_BASE_CHUNKS * _CHUNK) // _L + jnp.where(wid >= 28, 8, 0)

    cp_e = pltpu.async_copy(edge_hbm.at[:, pl.ds(c0, _MAX_EDGES)], edges_v, sem)

    with jax.named_scope("ph_srel_bcast"):

        @pl.when(sid == 0)
        def _():
            pltpu.sync_copy(srel_hbm.at[0], srel_sh)

        plsc.subcore_barrier()
        pltpu.sync_copy(srel_sh, srel_v)

    zero16 = jnp.zeros((_L,), jnp.float32)

    with jax.named_scope("ph_zero"):

        @plsc.parallel_loop(0, N_NODES // _L, unroll=8)
        def zero_step(i):
            acc_v[pl.ds(i * _L, _L)] = zero16

    with jax.named_scope("ph_dma_wait"):
        cp_e.wait()

    with jax.named_scope("ph_edges"):

        @plsc.parallel_loop(0, nvec, unroll=8)
        def edge_step(k):
            sl = pl.ds(k * _L, _L)
            vals = plsc.load_gather(srel_v, [edges_v[0, sl]])
            plsc.addupdate_scatter(acc_v, [edges_v[1, sl]], vals)

    with jax.named_scope("ph_out"):
        pltpu.sync_copy(acc_v, out_hbm.at[wid])


_edge_kernel = functools.partial(
    pl.kernel,
    mesh=plsc.VectorSubcoreMesh(core_axis_name="c", subcore_axis_name="s"),
    compiler_params=pltpu.CompilerParams(needs_layout_passes=False),
    out_type=jax.ShapeDtypeStruct((_NW, N_NODES), jnp.float32),
    scratch_types=[
        pltpu.VMEM_SHARED((N_NODES,), jnp.float32),
        pltpu.VMEM((N_NODES,), jnp.float32),
        pltpu.VMEM((2, _MAX_EDGES), jnp.int32),
        pltpu.VMEM((N_NODES,), jnp.float32),
        pltpu.SemaphoreType.DMA,
    ],
)(_edge_body)


def _combine_body(p_ref, base_ref, out_ref):
    out_ref[...] = jnp.sum(p_ref[...], axis=0, keepdims=True) + base_ref[...]


def kernel(x, edge_index, W_rel, b_rel, W_root):
    edges = edge_index.astype(jnp.int32)
    srel, base = pl.pallas_call(
        _matvec_body,
        out_shape=[
            jax.ShapeDtypeStruct((1, N_NODES), jnp.float32),
            jax.ShapeDtypeStruct((1, N_NODES), jnp.float32),
        ],
    )(x, W_rel.reshape(1, D), W_root.reshape(1, D), b_rel.reshape(1, 1))

    partials = _edge_kernel(srel, edges)

    out_row = pl.pallas_call(
        _combine_body,
        out_shape=jax.ShapeDtypeStruct((1, N_NODES), jnp.float32),
    )(partials, base)
    return out_row.reshape(N_NODES, 1)
